# initial kernel scaffold (unmeasured)
import jax
import jax.numpy as jnp
from jax import lax
from jax.experimental import pallas as pl
from jax.experimental.pallas import tpu as pltpu

N_DEV = 4


def kernel(t, W):
    M, K = t.shape
    _, N = W.shape
    C = M // N_DEV
    RT = 512
    f32 = jnp.float32
    bf16 = jnp.bfloat16

    def body(t_hbm, w_hbm, out_hbm, stage, wb, s_bf16, out_vmem,
             recv_hbm, ag_hbm, cp_sems, out_cp_sem,
             a2a_send_sems, a2a_recv_sems, ag_send_sems, ag_recv_sems):
        d = lax.axis_index("i")

        barrier = pltpu.get_barrier_semaphore()
        for o in range(1, N_DEV):
            peer = lax.rem(d + o, N_DEV)
            pl.semaphore_signal(barrier, inc=1, device_id=(peer,),
                                device_id_type=pl.DeviceIdType.MESH)
        pl.semaphore_wait(barrier, N_DEV - 1)

        a2a_sends = []
        for o in range(1, N_DEV):
            peer = lax.rem(d + o, N_DEV)
            slot = N_DEV - 1 - o
            rdma = pltpu.make_async_remote_copy(
                src_ref=t_hbm.at[pl.ds(peer * C, C), :],
                dst_ref=recv_hbm.at[slot],
                send_sem=a2a_send_sems.at[o - 1],
                recv_sem=a2a_recv_sems.at[slot],
                device_id=(peer,),
                device_id_type=pl.DeviceIdType.MESH,
            )
            rdma.start()
            a2a_sends.append(rdma)

        for kt in range(K // RT):
            cp = pltpu.make_async_copy(
                w_hbm.at[pl.ds(kt * RT, RT), :], stage.at[0], cp_sems.at[0])
            cp.start()
            cp.wait()
            wb[pl.ds(kt * RT, RT), :] = stage[0].astype(bf16)

        for o in range(1, N_DEV):
            slot = N_DEV - 1 - o
            recv = pltpu.make_async_remote_copy(
                src_ref=recv_hbm.at[slot],
                dst_ref=recv_hbm.at[slot],
                send_sem=a2a_send_sems.at[o - 1],
                recv_sem=a2a_recv_sems.at[slot],
                device_id=(d,),
                device_id_type=pl.DeviceIdType.MESH,
            )
            recv.wait_recv()

        for rt in range(C // RT):
            cps = [pltpu.make_async_copy(
                t_hbm.at[pl.ds(d * C + rt * RT, RT), :], stage.at[0],
                cp_sems.at[0])]
            for j in range(N_DEV - 1):
                cps.append(pltpu.make_async_copy(
                    recv_hbm.at[j, pl.ds(rt * RT, RT), :], stage.at[1 + j],
                    cp_sems.at[1 + j]))
            for cp in cps:
                cp.start()
            for cp in cps:
                cp.wait()
            acc = stage[0] + stage[1] + stage[2] + stage[3]
            s_bf16[pl.ds(rt * RT, RT), :] = acc.astype(bf16)

        out_vmem[:, :] = lax.dot_general(
            s_bf16[:, :], wb[:, :], (((1,), (0,)), ((), ())),
            preferred_element_type=f32)

        out_cp = pltpu.make_async_copy(
            out_vmem, out_hbm.at[pl.ds(d * C, C), :], out_cp_sem)
        out_cp.start()

        ag_sends = []
        for o in range(1, N_DEV):
            peer = lax.rem(d + o, N_DEV)
            slot = N_DEV - 1 - o
            rdma = pltpu.make_async_remote_copy(
                src_ref=out_vmem,
                dst_ref=ag_hbm.at[slot],
                send_sem=ag_send_sems.at[o - 1],
                recv_sem=ag_recv_sems.at[slot],
                device_id=(peer,),
                device_id_type=pl.DeviceIdType.MESH,
            )
            rdma.start()
            ag_sends.append(rdma)

        ag_cps = []
        for o in range(1, N_DEV):
            slot = N_DEV - 1 - o
            sender = lax.rem(d - o + N_DEV, N_DEV)
            recv = pltpu.make_async_remote_copy(
                src_ref=ag_hbm.at[slot],
                dst_ref=ag_hbm.at[slot],
                send_sem=ag_send_sems.at[o - 1],
                recv_sem=ag_recv_sems.at[slot],
                device_id=(d,),
                device_id_type=pl.DeviceIdType.MESH,
            )
            recv.wait_recv()
            cp = pltpu.make_async_copy(
                ag_hbm.at[slot], out_hbm.at[pl.ds(sender * C, C), :],
                cp_sems.at[o - 1])
            cp.start()
            ag_cps.append(cp)

        for rdma in a2a_sends:
            rdma.wait_send()
        for rdma in ag_sends:
            rdma.wait_send()
        for cp in ag_cps:
            cp.wait()
        out_cp.wait()

    return pl.pallas_call(
        body,
        out_shape=jax.ShapeDtypeStruct((M, N), f32),
        in_specs=[pl.BlockSpec(memory_space=pltpu.HBM),
                  pl.BlockSpec(memory_space=pltpu.HBM)],
        out_specs=pl.BlockSpec(memory_space=pltpu.HBM),
        scratch_shapes=[
            pltpu.VMEM((N_DEV, RT, K), f32),
            pltpu.VMEM((K, N), bf16),
            pltpu.VMEM((C, K), bf16),
            pltpu.VMEM((C, N), f32),
            pltpu.HBM((N_DEV - 1, C, K), f32),
            pltpu.HBM((N_DEV - 1, C, N), f32),
            pltpu.SemaphoreType.DMA((N_DEV,)),
            pltpu.SemaphoreType.DMA,
            pltpu.SemaphoreType.DMA((N_DEV - 1,)),
            pltpu.SemaphoreType.DMA((N_DEV - 1,)),
            pltpu.SemaphoreType.DMA((N_DEV - 1,)),
            pltpu.SemaphoreType.DMA((N_DEV - 1,)),
        ],
        compiler_params=pltpu.CompilerParams(collective_id=0),
    )(t, W)


# baseline (device time: 2172198 ns/iter reference)
import jax
import jax.numpy as jnp
from jax import lax
from jax.experimental import pallas as pl
from jax.experimental.pallas import tpu as pltpu

N_DEV = 4


def kernel(t, W):
    M, K = t.shape
    _, N = W.shape
    C = M // N_DEV
    RT = 256
    f32 = jnp.float32
    bf16 = jnp.bfloat16

    def body(t_hbm, w_hbm, out_hbm, recv_hbm, ag_hbm,
             stage, wb, s_bf16, out_vmem, cp_sems, out_cp_sem,
             a2a_send_sems, a2a_recv_sems, ag_send_sems, ag_recv_sems):
        d = lax.axis_index("i")

        barrier = pltpu.get_barrier_semaphore()
        for o in range(1, N_DEV):
            peer = lax.rem(d + o, N_DEV)
            pl.semaphore_signal(barrier, inc=1, device_id=(peer,),
                                device_id_type=pl.DeviceIdType.MESH)
        pl.semaphore_wait(barrier, N_DEV - 1)

        a2a_sends = []
        for o in range(1, N_DEV):
            peer = lax.rem(d + o, N_DEV)
            slot = N_DEV - 1 - o
            rdma = pltpu.make_async_remote_copy(
                src_ref=t_hbm.at[pl.ds(peer * C, C), :],
                dst_ref=recv_hbm.at[slot],
                send_sem=a2a_send_sems.at[o - 1],
                recv_sem=a2a_recv_sems.at[slot],
                device_id=(peer,),
                device_id_type=pl.DeviceIdType.MESH,
            )
            rdma.start()
            a2a_sends.append(rdma)

        for kt in range(K // RT):
            cp = pltpu.make_async_copy(
                w_hbm.at[pl.ds(kt * RT, RT), :], stage.at[0], cp_sems.at[0])
            cp.start()
            cp.wait()
            wb[pl.ds(kt * RT, RT), :] = stage[0].astype(bf16)

        for o in range(1, N_DEV):
            slot = N_DEV - 1 - o
            recv = pltpu.make_async_remote_copy(
                src_ref=recv_hbm.at[slot],
                dst_ref=recv_hbm.at[slot],
                send_sem=a2a_send_sems.at[o - 1],
                recv_sem=a2a_recv_sems.at[slot],
                device_id=(d,),
                device_id_type=pl.DeviceIdType.MESH,
            )
            recv.wait_recv()

        for rt in range(C // RT):
            cps = [pltpu.make_async_copy(
                t_hbm.at[pl.ds(d * C + rt * RT, RT), :], stage.at[0],
                cp_sems.at[0])]
            for j in range(N_DEV - 1):
                cps.append(pltpu.make_async_copy(
                    recv_hbm.at[j, pl.ds(rt * RT, RT), :], stage.at[1 + j],
                    cp_sems.at[1 + j]))
            for cp in cps:
                cp.start()
            for cp in cps:
                cp.wait()
            acc = stage[0] + stage[1] + stage[2] + stage[3]
            s_bf16[pl.ds(rt * RT, RT), :] = acc.astype(bf16)

        out_vmem[:, :] = lax.dot_general(
            s_bf16[:, :], wb[:, :], (((1,), (0,)), ((), ())),
            preferred_element_type=f32)

        out_cp = pltpu.make_async_copy(
            out_vmem, out_hbm.at[pl.ds(d * C, C), :], out_cp_sem)
        out_cp.start()

        ag_sends = []
        for o in range(1, N_DEV):
            peer = lax.rem(d + o, N_DEV)
            slot = N_DEV - 1 - o
            rdma = pltpu.make_async_remote_copy(
                src_ref=out_vmem,
                dst_ref=ag_hbm.at[slot],
                send_sem=ag_send_sems.at[o - 1],
                recv_sem=ag_recv_sems.at[slot],
                device_id=(peer,),
                device_id_type=pl.DeviceIdType.MESH,
            )
            rdma.start()
            ag_sends.append(rdma)

        ag_cps = []
        for o in range(1, N_DEV):
            slot = N_DEV - 1 - o
            sender = lax.rem(d - o + N_DEV, N_DEV)
            recv = pltpu.make_async_remote_copy(
                src_ref=ag_hbm.at[slot],
                dst_ref=ag_hbm.at[slot],
                send_sem=ag_send_sems.at[o - 1],
                recv_sem=ag_recv_sems.at[slot],
                device_id=(d,),
                device_id_type=pl.DeviceIdType.MESH,
            )
            recv.wait_recv()
            cp = pltpu.make_async_copy(
                ag_hbm.at[slot], out_hbm.at[pl.ds(sender * C, C), :],
                cp_sems.at[o - 1])
            cp.start()
            ag_cps.append(cp)

        for rdma in a2a_sends:
            rdma.wait_send()
        for rdma in ag_sends:
            rdma.wait_send()
        for cp in ag_cps:
            cp.wait()
        out_cp.wait()

    out, _, _ = pl.pallas_call(
        body,
        out_shape=[
            jax.ShapeDtypeStruct((M, N), f32),
            jax.ShapeDtypeStruct((N_DEV - 1, C, K), f32),
            jax.ShapeDtypeStruct((N_DEV - 1, C, N), f32),
        ],
        in_specs=[pl.BlockSpec(memory_space=pltpu.HBM),
                  pl.BlockSpec(memory_space=pltpu.HBM)],
        out_specs=[pl.BlockSpec(memory_space=pltpu.HBM)] * 3,
        scratch_shapes=[
            pltpu.VMEM((N_DEV, RT, K), f32),
            pltpu.VMEM((K, N), bf16),
            pltpu.VMEM((C, K), bf16),
            pltpu.VMEM((C, N), f32),
            pltpu.SemaphoreType.DMA((N_DEV,)),
            pltpu.SemaphoreType.DMA,
            pltpu.SemaphoreType.DMA((N_DEV - 1,)),
            pltpu.SemaphoreType.DMA((N_DEV - 1,)),
            pltpu.SemaphoreType.DMA((N_DEV - 1,)),
            pltpu.SemaphoreType.DMA((N_DEV - 1,)),
        ],
        compiler_params=pltpu.CompilerParams(
            collective_id=0, vmem_limit_bytes=60 * 1024 * 1024),
    )(t, W)
    return out


# device time: 420669 ns/iter; 5.1637x vs baseline; 5.1637x over previous
import jax
import jax.numpy as jnp
from jax import lax
from jax.experimental import pallas as pl
from jax.experimental.pallas import tpu as pltpu

N_DEV = 4


def kernel(t, W):
    M, K = t.shape
    _, N = W.shape
    C = M // N_DEV
    RT = 512
    NRT = C // RT
    f32 = jnp.float32
    bf16 = jnp.bfloat16

    def body(t_hbm, w_hbm, out_hbm, tb_hbm, rs_hbm, ag_hbm,
             stage, bstage, wstage, wb, out_bf16, cp_sems, ocp_sems,
             a2a_send_sems, a2a_recv_sems, ag_send_sems, ag_recv_sems):
        d = lax.axis_index("i")

        barrier = pltpu.get_barrier_semaphore()
        for o in range(1, N_DEV):
            peer = lax.rem(d + o, N_DEV)
            pl.semaphore_signal(barrier, inc=1, device_id=(peer,),
                                device_id_type=pl.DeviceIdType.MESH)
        pl.semaphore_wait(barrier, N_DEV - 1)

        tiles = [(o, rt) for rt in range(NRT) for o in range(1, N_DEV)]

        def start_in(i):
            o, rt = tiles[i]
            peer = lax.rem(d + o, N_DEV)
            cp = pltpu.make_async_copy(
                t_hbm.at[pl.ds(peer * C + rt * RT, RT), :],
                stage.at[i % 2], cp_sems.at[i % 2])
            cp.start()
            return cp

        a2a_sends = []
        pending = start_in(0)
        for i, (o, rt) in enumerate(tiles):
            pending.wait()
            if i + 1 < len(tiles):
                nxt = start_in(i + 1)
            bstage[3, :, :] = stage[i % 2].astype(bf16)
            ocp = pltpu.make_async_copy(
                bstage.at[3], tb_hbm.at[o - 1, pl.ds(rt * RT, RT), :],
                ocp_sems.at[0])
            ocp.start()
            ocp.wait()
            peer = lax.rem(d + o, N_DEV)
            rdma = pltpu.make_async_remote_copy(
                src_ref=tb_hbm.at[o - 1, pl.ds(rt * RT, RT), :],
                dst_ref=rs_hbm.at[N_DEV - 1 - o, pl.ds(rt * RT, RT), :],
                send_sem=a2a_send_sems.at[o - 1, rt],
                recv_sem=a2a_recv_sems.at[N_DEV - 1 - o, rt],
                device_id=(peer,),
                device_id_type=pl.DeviceIdType.MESH,
            )
            rdma.start()
            a2a_sends.append(rdma)
            if i + 1 < len(tiles):
                pending = nxt

        wcps = [pltpu.make_async_copy(
            w_hbm.at[pl.ds(kt * RT, RT), :], wstage.at[kt % 2],
            ocp_sems.at[kt % 2]) for kt in range(K // RT)]
        wcps[0].start()
        for kt in range(K // RT):
            wcps[kt].wait()
            if kt + 1 < len(wcps):
                wcps[kt + 1].start()
            wb[pl.ds(kt * RT, RT), :] = wstage[kt % 2].astype(bf16)

        ag_sends = []
        for rt in range(NRT):
            for j in range(N_DEV - 1):
                recv = pltpu.make_async_remote_copy(
                    src_ref=rs_hbm.at[j, pl.ds(rt * RT, RT), :],
                    dst_ref=rs_hbm.at[j, pl.ds(rt * RT, RT), :],
                    send_sem=a2a_send_sems.at[j, rt],
                    recv_sem=a2a_recv_sems.at[j, rt],
                    device_id=(d,),
                    device_id_type=pl.DeviceIdType.MESH,
                )
                recv.wait_recv()
            cps = [pltpu.make_async_copy(
                t_hbm.at[pl.ds(d * C + rt * RT, RT), :], stage.at[0],
                cp_sems.at[0])]
            for j in range(N_DEV - 1):
                cps.append(pltpu.make_async_copy(
                    rs_hbm.at[j, pl.ds(rt * RT, RT), :], bstage.at[j],
                    cp_sems.at[1 + j]))
            for cp in cps:
                cp.start()
            for cp in cps:
                cp.wait()
            s_sub = (stage[0] + bstage[0].astype(f32) +
                     bstage[1].astype(f32) + bstage[2].astype(f32))
            acc = lax.dot_general(
                s_sub.astype(bf16), wb[:, :], (((1,), (0,)), ((), ())),
                preferred_element_type=f32)
            stage[1, :, :] = acc
            own = pltpu.make_async_copy(
                stage.at[1],
                out_hbm.at[pl.ds(d * C + rt * RT, RT), :],
                ocp_sems.at[1])
            own.start()
            out_bf16[pl.ds(rt * RT, RT), :] = acc.astype(bf16)
            for o in range(1, N_DEV):
                peer = lax.rem(d + o, N_DEV)
                rdma = pltpu.make_async_remote_copy(
                    src_ref=out_bf16.at[pl.ds(rt * RT, RT), :],
                    dst_ref=ag_hbm.at[N_DEV - 1 - o, pl.ds(rt * RT, RT), :],
                    send_sem=ag_send_sems.at[o - 1, rt],
                    recv_sem=ag_recv_sems.at[N_DEV - 1 - o, rt],
                    device_id=(peer,),
                    device_id_type=pl.DeviceIdType.MESH,
                )
                rdma.start()
                ag_sends.append(rdma)
            own.wait()

        drain = [(o, rt) for rt in range(NRT) for o in range(1, N_DEV)]

        def start_drain_in(k):
            o, rt = drain[k]
            slot = N_DEV - 1 - o
            recv = pltpu.make_async_remote_copy(
                src_ref=ag_hbm.at[slot, pl.ds(rt * RT, RT), :],
                dst_ref=ag_hbm.at[slot, pl.ds(rt * RT, RT), :],
                send_sem=ag_send_sems.at[o - 1, rt],
                recv_sem=ag_recv_sems.at[slot, rt],
                device_id=(d,),
                device_id_type=pl.DeviceIdType.MESH,
            )
            recv.wait_recv()
            cp = pltpu.make_async_copy(
                ag_hbm.at[slot, pl.ds(rt * RT, RT), :], bstage.at[k % 2],
                cp_sems.at[k % 2])
            cp.start()
            return cp

        prev_out = [None, None]
        pend = start_drain_in(0)
        for k, (o, rt) in enumerate(drain):
            pend.wait()
            if k + 1 < len(drain):
                nxt = start_drain_in(k + 1)
            if prev_out[k % 2] is not None:
                prev_out[k % 2].wait()
            stage[k % 2, :, :] = bstage[k % 2].astype(f32)
            sender = lax.rem(d - o + N_DEV, N_DEV)
            ocp = pltpu.make_async_copy(
                stage.at[k % 2],
                out_hbm.at[pl.ds(sender * C + rt * RT, RT), :],
                ocp_sems.at[k % 2])
            ocp.start()
            prev_out[k % 2] = ocp
            if k + 1 < len(drain):
                pend = nxt
        for p in prev_out:
            if p is not None:
                p.wait()

        for rdma in a2a_sends:
            rdma.wait_send()
        for rdma in ag_sends:
            rdma.wait_send()

    out, _, _, _ = pl.pallas_call(
        body,
        out_shape=[
            jax.ShapeDtypeStruct((M, N), f32),
            jax.ShapeDtypeStruct((N_DEV - 1, C, K), bf16),
            jax.ShapeDtypeStruct((N_DEV - 1, C, K), bf16),
            jax.ShapeDtypeStruct((N_DEV - 1, C, N), bf16),
        ],
        in_specs=[pl.BlockSpec(memory_space=pltpu.HBM),
                  pl.BlockSpec(memory_space=pltpu.HBM)],
        out_specs=[pl.BlockSpec(memory_space=pltpu.HBM)] * 4,
        scratch_shapes=[
            pltpu.VMEM((2, RT, K), f32),
            pltpu.VMEM((4, RT, K), bf16),
            pltpu.VMEM((2, RT, K), f32),
            pltpu.VMEM((K, N), bf16),
            pltpu.VMEM((C, N), bf16),
            pltpu.SemaphoreType.DMA((N_DEV,)),
            pltpu.SemaphoreType.DMA((2,)),
            pltpu.SemaphoreType.DMA((N_DEV - 1, NRT)),
            pltpu.SemaphoreType.DMA((N_DEV - 1, NRT)),
            pltpu.SemaphoreType.DMA((N_DEV - 1, NRT)),
            pltpu.SemaphoreType.DMA((N_DEV - 1, NRT)),
        ],
        compiler_params=pltpu.CompilerParams(
            collective_id=0, vmem_limit_bytes=60 * 1024 * 1024),
    )(t, W)
    return out


# device time: 419836 ns/iter; 5.1739x vs baseline; 1.0020x over previous
import jax
import jax.numpy as jnp
from jax import lax
from jax.experimental import pallas as pl
from jax.experimental.pallas import tpu as pltpu

N_DEV = 4


def kernel(t, W):
    M, K = t.shape
    _, N = W.shape
    C = M // N_DEV
    RT = 512
    NRT = C // RT
    f32 = jnp.float32
    bf16 = jnp.bfloat16

    def body(t_hbm, w_hbm, out_hbm, tb_hbm, rs_hbm, ag_hbm,
             stage, bstage, wstage, wb, out_bf16, cp_sems, ocp_sems,
             a2a_send_sems, a2a_recv_sems, ag_send_sems, ag_recv_sems):
        d = lax.axis_index("i")

        barrier = pltpu.get_barrier_semaphore()
        for o in range(1, N_DEV):
            peer = lax.rem(d + o, N_DEV)
            pl.semaphore_signal(barrier, inc=1, device_id=(peer,),
                                device_id_type=pl.DeviceIdType.MESH)
        pl.semaphore_wait(barrier, N_DEV - 1)

        tiles = [(o, rt) for rt in range(NRT) for o in range(1, N_DEV)]

        def start_in(i):
            o, rt = tiles[i]
            peer = lax.rem(d + o, N_DEV)
            cp = pltpu.make_async_copy(
                t_hbm.at[pl.ds(peer * C + rt * RT, RT), :],
                stage.at[i % 2], cp_sems.at[i % 2])
            cp.start()
            return cp

        a2a_sends = []
        pending = start_in(0)
        for i, (o, rt) in enumerate(tiles):
            pending.wait()
            if i + 1 < len(tiles):
                nxt = start_in(i + 1)
            bstage[3, :, :] = stage[i % 2].astype(bf16)
            ocp = pltpu.make_async_copy(
                bstage.at[3], tb_hbm.at[o - 1, pl.ds(rt * RT, RT), :],
                ocp_sems.at[0])
            ocp.start()
            ocp.wait()
            peer = lax.rem(d + o, N_DEV)
            rdma = pltpu.make_async_remote_copy(
                src_ref=tb_hbm.at[o - 1, pl.ds(rt * RT, RT), :],
                dst_ref=rs_hbm.at[N_DEV - 1 - o, pl.ds(rt * RT, RT), :],
                send_sem=a2a_send_sems.at[o - 1, rt],
                recv_sem=a2a_recv_sems.at[N_DEV - 1 - o, rt],
                device_id=(peer,),
                device_id_type=pl.DeviceIdType.MESH,
            )
            rdma.start()
            a2a_sends.append(rdma)
            if i + 1 < len(tiles):
                pending = nxt

        wcps = [pltpu.make_async_copy(
            w_hbm.at[pl.ds(kt * RT, RT), :], wstage.at[kt % 2],
            ocp_sems.at[kt % 2]) for kt in range(K // RT)]
        wcps[0].start()
        for kt in range(K // RT):
            wcps[kt].wait()
            if kt + 1 < len(wcps):
                wcps[kt + 1].start()
            wb[pl.ds(kt * RT, RT), :] = wstage[kt % 2].astype(bf16)

        drain = [(o, rt) for rt in range(NRT) for o in range(1, N_DEV)]
        prev_drain_out = [None, None]

        def drain_tile(k):
            o, rt = drain[k]
            slot = N_DEV - 1 - o
            sender = lax.rem(d - o + N_DEV, N_DEV)
            recv = pltpu.make_async_remote_copy(
                src_ref=ag_hbm.at[slot, pl.ds(rt * RT, RT), :],
                dst_ref=ag_hbm.at[slot, pl.ds(rt * RT, RT), :],
                send_sem=ag_send_sems.at[o - 1, rt],
                recv_sem=ag_recv_sems.at[slot, rt],
                device_id=(d,),
                device_id_type=pl.DeviceIdType.MESH,
            )
            recv.wait_recv()
            icp = pltpu.make_async_copy(
                ag_hbm.at[slot, pl.ds(rt * RT, RT), :], bstage.at[3],
                cp_sems.at[3])
            icp.start()
            icp.wait()
            w = k % 2
            if prev_drain_out[w] is not None:
                prev_drain_out[w].wait()
            wstage[w, :, :] = bstage[3].astype(f32)
            ocp = pltpu.make_async_copy(
                wstage.at[w],
                out_hbm.at[pl.ds(sender * C + rt * RT, RT), :],
                ocp_sems.at[2 + w])
            ocp.start()
            prev_drain_out[w] = ocp

        ag_sends = []
        prev_own = None
        for rt in range(NRT):
            for j in range(N_DEV - 1):
                recv = pltpu.make_async_remote_copy(
                    src_ref=rs_hbm.at[j, pl.ds(rt * RT, RT), :],
                    dst_ref=rs_hbm.at[j, pl.ds(rt * RT, RT), :],
                    send_sem=a2a_send_sems.at[j, rt],
                    recv_sem=a2a_recv_sems.at[j, rt],
                    device_id=(d,),
                    device_id_type=pl.DeviceIdType.MESH,
                )
                recv.wait_recv()
            cps = [pltpu.make_async_copy(
                t_hbm.at[pl.ds(d * C + rt * RT, RT), :], stage.at[0],
                cp_sems.at[0])]
            for j in range(N_DEV - 1):
                cps.append(pltpu.make_async_copy(
                    rs_hbm.at[j, pl.ds(rt * RT, RT), :], bstage.at[j],
                    cp_sems.at[1 + j]))
            for cp in cps:
                cp.start()
            for cp in cps:
                cp.wait()
            s_sub = (stage[0] + bstage[0].astype(f32) +
                     bstage[1].astype(f32) + bstage[2].astype(f32))
            acc = lax.dot_general(
                s_sub.astype(bf16), wb[:, :], (((1,), (0,)), ((), ())),
                preferred_element_type=f32)
            if prev_own is not None:
                prev_own.wait()
            stage[1, :, :] = acc
            own = pltpu.make_async_copy(
                stage.at[1],
                out_hbm.at[pl.ds(d * C + rt * RT, RT), :],
                ocp_sems.at[1])
            own.start()
            prev_own = own
            out_bf16[pl.ds(rt * RT, RT), :] = acc.astype(bf16)
            for o in range(1, N_DEV):
                peer = lax.rem(d + o, N_DEV)
                rdma = pltpu.make_async_remote_copy(
                    src_ref=out_bf16.at[pl.ds(rt * RT, RT), :],
                    dst_ref=ag_hbm.at[N_DEV - 1 - o, pl.ds(rt * RT, RT), :],
                    send_sem=ag_send_sems.at[o - 1, rt],
                    recv_sem=ag_recv_sems.at[N_DEV - 1 - o, rt],
                    device_id=(peer,),
                    device_id_type=pl.DeviceIdType.MESH,
                )
                rdma.start()
                ag_sends.append(rdma)
            if rt >= 1:
                for j in range(3):
                    drain_tile(3 * (rt - 1) + j)

        for k in range(3 * (NRT - 1), 3 * NRT):
            drain_tile(k)
        for p in prev_drain_out:
            if p is not None:
                p.wait()
        prev_own.wait()

        for rdma in a2a_sends:
            rdma.wait_send()
        for rdma in ag_sends:
            rdma.wait_send()

    out, _, _, _ = pl.pallas_call(
        body,
        out_shape=[
            jax.ShapeDtypeStruct((M, N), f32),
            jax.ShapeDtypeStruct((N_DEV - 1, C, K), bf16),
            jax.ShapeDtypeStruct((N_DEV - 1, C, K), bf16),
            jax.ShapeDtypeStruct((N_DEV - 1, C, N), bf16),
        ],
        in_specs=[pl.BlockSpec(memory_space=pltpu.HBM),
                  pl.BlockSpec(memory_space=pltpu.HBM)],
        out_specs=[pl.BlockSpec(memory_space=pltpu.HBM)] * 4,
        scratch_shapes=[
            pltpu.VMEM((2, RT, K), f32),
            pltpu.VMEM((4, RT, K), bf16),
            pltpu.VMEM((2, RT, K), f32),
            pltpu.VMEM((K, N), bf16),
            pltpu.VMEM((C, N), bf16),
            pltpu.SemaphoreType.DMA((N_DEV,)),
            pltpu.SemaphoreType.DMA((4,)),
            pltpu.SemaphoreType.DMA((N_DEV - 1, NRT)),
            pltpu.SemaphoreType.DMA((N_DEV - 1, NRT)),
            pltpu.SemaphoreType.DMA((N_DEV - 1, NRT)),
            pltpu.SemaphoreType.DMA((N_DEV - 1, NRT)),
        ],
        compiler_params=pltpu.CompilerParams(
            collective_id=0, vmem_limit_bytes=60 * 1024 * 1024),
    )(t, W)
    return out
